# baseline (speedup 1.0000x reference)
import jax
import jax.numpy as jnp
from jax.experimental import pallas as pl
from jax.experimental.pallas import tpu as pltpu

N = 10000
D = 512
TILE_M1 = 200
TILE_M = 200
TILE_T = 1000


def _xform_kernel(h_ref, w_ref, b_ref, t_ref):
    acc = jnp.dot(h_ref[...], w_ref[...], preferred_element_type=jnp.float32)
    t_ref[...] = (acc + b_ref[...]).astype(jnp.bfloat16)


def _agg_cast_kernel(hg_ref, t_ref, out_ref, hg16_ref):
    hg16 = hg_ref[...].astype(jnp.bfloat16)
    hg16_ref[...] = hg16
    acc = jnp.dot(hg16, t_ref[...], preferred_element_type=jnp.float32)
    out_ref[...] = jnp.maximum(acc, 0.0)


def _l23_kernel(hg16_ref, t2_ref, w3_ref, b3_ref, out_ref, t_ref):
    l = pl.program_id(0)
    i = pl.program_id(1)

    @pl.when((l == 0) & (i == 0))
    def _load_t2():
        t_ref[...] = t2_ref[...]

    @pl.when((l == 1) & (i == 0))
    def _compute_t3():
        def body(c, _):
            rows = pl.ds(c * TILE_T, TILE_T)
            acc = jnp.dot(out_ref[rows, :], w3_ref[...],
                          preferred_element_type=jnp.float32)
            t_ref[rows, :] = (acc + b3_ref[...]).astype(jnp.bfloat16)
            return 0
        jax.lax.fori_loop(0, N // TILE_T, body, 0)

    acc = jnp.dot(hg16_ref[...], t_ref[...],
                  preferred_element_type=jnp.float32)
    out_ref[pl.ds(i * TILE_M, TILE_M), :] = jnp.maximum(acc, 0.0)


def _xform(h, W, b):
    return pl.pallas_call(
        _xform_kernel,
        grid=(N // TILE_T,),
        in_specs=[
            pl.BlockSpec((TILE_T, D), lambda i: (i, 0)),
            pl.BlockSpec((D, D), lambda i: (0, 0)),
            pl.BlockSpec((1, D), lambda i: (0, 0)),
        ],
        out_specs=pl.BlockSpec((TILE_T, D), lambda i: (i, 0)),
        out_shape=jax.ShapeDtypeStruct((N, D), jnp.bfloat16),
        compiler_params=pltpu.CompilerParams(
            dimension_semantics=("parallel",)),
    )(h, W, b.reshape(1, D))


def kernel(x, hg, W1, b1, W2, b2, W3, b3):
    t1 = _xform(x, W1, b1)

    h1, hg16 = pl.pallas_call(
        _agg_cast_kernel,
        grid=(N // TILE_M1,),
        in_specs=[
            pl.BlockSpec((TILE_M1, N), lambda i: (i, 0)),
            pl.BlockSpec((N, D), lambda i: (0, 0)),
        ],
        out_specs=[
            pl.BlockSpec((TILE_M1, D), lambda i: (i, 0)),
            pl.BlockSpec((TILE_M1, N), lambda i: (i, 0)),
        ],
        out_shape=[
            jax.ShapeDtypeStruct((N, D), jnp.float32),
            jax.ShapeDtypeStruct((N, N), jnp.bfloat16),
        ],
        compiler_params=pltpu.CompilerParams(
            dimension_semantics=("parallel",)),
    )(hg, t1)

    t2 = _xform(h1, W2, b2)

    return pl.pallas_call(
        _l23_kernel,
        grid=(2, N // TILE_M),
        in_specs=[
            pl.BlockSpec((TILE_M, N), lambda l, i: (i, 0)),
            pl.BlockSpec((N, D), lambda l, i: (0, 0)),
            pl.BlockSpec((D, D), lambda l, i: (0, 0)),
            pl.BlockSpec((1, D), lambda l, i: (0, 0)),
        ],
        out_specs=pl.BlockSpec((N, D), lambda l, i: (0, 0)),
        out_shape=jax.ShapeDtypeStruct((N, D), jnp.float32),
        scratch_shapes=[pltpu.VMEM((N, D), jnp.bfloat16)],
        compiler_params=pltpu.CompilerParams(
            dimension_semantics=("arbitrary", "arbitrary"),
            vmem_limit_bytes=100 * 1024 * 1024,
        ),
    )(hg16, t2, W3, b3.reshape(1, D))


# speedup vs baseline: 1.0251x; 1.0251x over previous
import jax
import jax.numpy as jnp
from jax.experimental import pallas as pl
from jax.experimental.pallas import tpu as pltpu

N = 10000
D = 512
TILE_M1 = 200
TILE_M = 400
TILE_T = 1000


def _xform_kernel(h_ref, w_ref, b_ref, t_ref):
    acc = jnp.dot(h_ref[...], w_ref[...], preferred_element_type=jnp.float32)
    t_ref[...] = (acc + b_ref[...]).astype(jnp.bfloat16)


def _agg_cast_kernel(hg_ref, t_ref, out_ref, hg16_ref):
    hg16 = hg_ref[...].astype(jnp.bfloat16)
    hg16_ref[...] = hg16
    acc = jnp.dot(hg16, t_ref[...], preferred_element_type=jnp.float32)
    out_ref[...] = jnp.maximum(acc, 0.0)


def _agg_kernel(hg16_ref, t_ref, out_ref):
    acc = jnp.dot(hg16_ref[...], t_ref[...],
                  preferred_element_type=jnp.float32)
    out_ref[...] = jnp.maximum(acc, 0.0)


def _xform(h, W, b):
    return pl.pallas_call(
        _xform_kernel,
        grid=(N // TILE_T,),
        in_specs=[
            pl.BlockSpec((TILE_T, D), lambda i: (i, 0)),
            pl.BlockSpec((D, D), lambda i: (0, 0)),
            pl.BlockSpec((1, D), lambda i: (0, 0)),
        ],
        out_specs=pl.BlockSpec((TILE_T, D), lambda i: (i, 0)),
        out_shape=jax.ShapeDtypeStruct((N, D), jnp.bfloat16),
        compiler_params=pltpu.CompilerParams(
            dimension_semantics=("arbitrary",)),
    )(h, W, b.reshape(1, D))


def kernel(x, hg, W1, b1, W2, b2, W3, b3):
    t1 = _xform(x, W1, b1)
    h1, hg16 = pl.pallas_call(
        _agg_cast_kernel,
        grid=(N // TILE_M1,),
        in_specs=[
            pl.BlockSpec((TILE_M1, N), lambda i: (i, 0)),
            pl.BlockSpec((N, D), lambda i: (0, 0)),
        ],
        out_specs=[
            pl.BlockSpec((TILE_M1, D), lambda i: (i, 0)),
            pl.BlockSpec((TILE_M1, N), lambda i: (i, 0)),
        ],
        out_shape=[
            jax.ShapeDtypeStruct((N, D), jnp.float32),
            jax.ShapeDtypeStruct((N, N), jnp.bfloat16),
        ],
        compiler_params=pltpu.CompilerParams(
            dimension_semantics=("arbitrary",)),
    )(hg, t1)

    h = h1
    for W, b in ((W2, b2), (W3, b3)):
        t = _xform(h, W, b)
        h = pl.pallas_call(
            _agg_kernel,
            grid=(N // TILE_M,),
            in_specs=[
                pl.BlockSpec((TILE_M, N), lambda i: (i, 0)),
                pl.BlockSpec((N, D), lambda i: (0, 0)),
            ],
            out_specs=pl.BlockSpec((TILE_M, D), lambda i: (i, 0)),
            out_shape=jax.ShapeDtypeStruct((N, D), jnp.float32),
            compiler_params=pltpu.CompilerParams(
                dimension_semantics=("arbitrary",)),
        )(hg16, t)
    return h


# speedup vs baseline: 1.0427x; 1.0172x over previous
import jax
import jax.numpy as jnp
from jax.experimental import pallas as pl
from jax.experimental.pallas import tpu as pltpu

N = 10000
D = 512
TILE_M1 = 400
TILE_M = 1000
TILE_T = 1000


def _xform_kernel(h_ref, w_ref, b_ref, t_ref):
    acc = jnp.dot(h_ref[...], w_ref[...], preferred_element_type=jnp.float32)
    t_ref[...] = (acc + b_ref[...]).astype(jnp.bfloat16)


def _agg_cast_kernel(hg_ref, t_ref, out_ref, hg16_ref):
    hg16 = hg_ref[...].astype(jnp.bfloat16)
    hg16_ref[...] = hg16
    acc = jnp.dot(hg16, t_ref[...], preferred_element_type=jnp.float32)
    out_ref[...] = jnp.maximum(acc, 0.0)


def _agg_kernel(hg16_ref, t_ref, out_ref):
    acc = jnp.dot(hg16_ref[...], t_ref[...],
                  preferred_element_type=jnp.float32)
    out_ref[...] = jnp.maximum(acc, 0.0)


def _xform(h, W, b):
    return pl.pallas_call(
        _xform_kernel,
        grid=(N // TILE_T,),
        in_specs=[
            pl.BlockSpec((TILE_T, D), lambda i: (i, 0)),
            pl.BlockSpec((D, D), lambda i: (0, 0)),
            pl.BlockSpec((1, D), lambda i: (0, 0)),
        ],
        out_specs=pl.BlockSpec((TILE_T, D), lambda i: (i, 0)),
        out_shape=jax.ShapeDtypeStruct((N, D), jnp.bfloat16),
        compiler_params=pltpu.CompilerParams(
            dimension_semantics=("parallel",)),
    )(h, W, b.reshape(1, D))


def kernel(x, hg, W1, b1, W2, b2, W3, b3):
    t1 = _xform(x, W1, b1)
    h1, hg16 = pl.pallas_call(
        _agg_cast_kernel,
        grid=(N // TILE_M1,),
        in_specs=[
            pl.BlockSpec((TILE_M1, N), lambda i: (i, 0)),
            pl.BlockSpec((N, D), lambda i: (0, 0)),
        ],
        out_specs=[
            pl.BlockSpec((TILE_M1, D), lambda i: (i, 0)),
            pl.BlockSpec((TILE_M1, N), lambda i: (i, 0)),
        ],
        out_shape=[
            jax.ShapeDtypeStruct((N, D), jnp.float32),
            jax.ShapeDtypeStruct((N, N), jnp.bfloat16),
        ],
        compiler_params=pltpu.CompilerParams(
            dimension_semantics=("parallel",)),
    )(hg, t1)

    h = h1
    for W, b in ((W2, b2), (W3, b3)):
        t = _xform(h, W, b)
        h = pl.pallas_call(
            _agg_kernel,
            grid=(N // TILE_M,),
            in_specs=[
                pl.BlockSpec((TILE_M, N), lambda i: (i, 0)),
                pl.BlockSpec((N, D), lambda i: (0, 0)),
            ],
            out_specs=pl.BlockSpec((TILE_M, D), lambda i: (i, 0)),
            out_shape=jax.ShapeDtypeStruct((N, D), jnp.float32),
            compiler_params=pltpu.CompilerParams(
                dimension_semantics=("parallel",)),
        )(hg16, t)
    return h


# speedup vs baseline: 1.1006x; 1.0555x over previous
import jax
import jax.numpy as jnp
from jax.experimental import pallas as pl
from jax.experimental.pallas import tpu as pltpu

N = 10000
D = 512
TILE_M1 = 400
TILE_M = 1000
TILE_T = 1000


def _xform_kernel(h_ref, w_ref, b_ref, t_ref):
    acc = jnp.dot(h_ref[...], w_ref[...], preferred_element_type=jnp.float32)
    t_ref[...] = (acc + b_ref[...]).astype(jnp.bfloat16)


def _l1_kernel(hg_ref, t1_ref, w2_ref, b2_ref, hg16_ref, t2_ref):
    hg16 = hg_ref[...].astype(jnp.bfloat16)
    hg16_ref[...] = hg16
    acc = jnp.dot(hg16, t1_ref[...], preferred_element_type=jnp.float32)
    h1 = jnp.maximum(acc, 0.0).astype(jnp.bfloat16)
    t2 = jnp.dot(h1, w2_ref[...], preferred_element_type=jnp.float32)
    t2_ref[...] = (t2 + b2_ref[...]).astype(jnp.bfloat16)


def _l2_kernel(hg16_ref, t2_ref, w3_ref, b3_ref, t3_ref):
    acc = jnp.dot(hg16_ref[...], t2_ref[...],
                  preferred_element_type=jnp.float32)
    h2 = jnp.maximum(acc, 0.0).astype(jnp.bfloat16)
    t3 = jnp.dot(h2, w3_ref[...], preferred_element_type=jnp.float32)
    t3_ref[...] = (t3 + b3_ref[...]).astype(jnp.bfloat16)


def _l3_kernel(hg16_ref, t3_ref, out_ref):
    acc = jnp.dot(hg16_ref[...], t3_ref[...],
                  preferred_element_type=jnp.float32)
    out_ref[...] = jnp.maximum(acc, 0.0)


def kernel(x, hg, W1, b1, W2, b2, W3, b3):
    t1 = pl.pallas_call(
        _xform_kernel,
        grid=(N // TILE_T,),
        in_specs=[
            pl.BlockSpec((TILE_T, D), lambda i: (i, 0)),
            pl.BlockSpec((D, D), lambda i: (0, 0)),
            pl.BlockSpec((1, D), lambda i: (0, 0)),
        ],
        out_specs=pl.BlockSpec((TILE_T, D), lambda i: (i, 0)),
        out_shape=jax.ShapeDtypeStruct((N, D), jnp.bfloat16),
        compiler_params=pltpu.CompilerParams(
            dimension_semantics=("parallel",)),
    )(x, W1, b1.reshape(1, D))

    hg16, t2 = pl.pallas_call(
        _l1_kernel,
        grid=(N // TILE_M1,),
        in_specs=[
            pl.BlockSpec((TILE_M1, N), lambda i: (i, 0)),
            pl.BlockSpec((N, D), lambda i: (0, 0)),
            pl.BlockSpec((D, D), lambda i: (0, 0)),
            pl.BlockSpec((1, D), lambda i: (0, 0)),
        ],
        out_specs=[
            pl.BlockSpec((TILE_M1, N), lambda i: (i, 0)),
            pl.BlockSpec((TILE_M1, D), lambda i: (i, 0)),
        ],
        out_shape=[
            jax.ShapeDtypeStruct((N, N), jnp.bfloat16),
            jax.ShapeDtypeStruct((N, D), jnp.bfloat16),
        ],
        compiler_params=pltpu.CompilerParams(
            dimension_semantics=("parallel",)),
    )(hg, t1, W2.astype(jnp.bfloat16), b2.reshape(1, D))

    t3 = pl.pallas_call(
        _l2_kernel,
        grid=(N // TILE_M,),
        in_specs=[
            pl.BlockSpec((TILE_M, N), lambda i: (i, 0)),
            pl.BlockSpec((N, D), lambda i: (0, 0)),
            pl.BlockSpec((D, D), lambda i: (0, 0)),
            pl.BlockSpec((1, D), lambda i: (0, 0)),
        ],
        out_specs=pl.BlockSpec((TILE_M, D), lambda i: (i, 0)),
        out_shape=jax.ShapeDtypeStruct((N, D), jnp.bfloat16),
        compiler_params=pltpu.CompilerParams(
            dimension_semantics=("parallel",)),
    )(hg16, t2, W3.astype(jnp.bfloat16), b3.reshape(1, D))

    return pl.pallas_call(
        _l3_kernel,
        grid=(N // TILE_M,),
        in_specs=[
            pl.BlockSpec((TILE_M, N), lambda i: (i, 0)),
            pl.BlockSpec((N, D), lambda i: (0, 0)),
        ],
        out_specs=pl.BlockSpec((TILE_M, D), lambda i: (i, 0)),
        out_shape=jax.ShapeDtypeStruct((N, D), jnp.float32),
        compiler_params=pltpu.CompilerParams(
            dimension_semantics=("parallel",)),
    )(hg16, t3)


# speedup vs baseline: 1.1620x; 1.0558x over previous
import jax
import jax.numpy as jnp
from jax.experimental import pallas as pl
from jax.experimental.pallas import tpu as pltpu

N = 10000
D = 512
TILE_M = 400
TILE_T = 1000


def _xform_kernel(h_ref, w_ref, b_ref, t_ref):
    acc = jnp.dot(h_ref[...], w_ref[...], preferred_element_type=jnp.float32)
    t_ref[...] = (acc + b_ref[...]).astype(jnp.bfloat16)


def _mid_kernel(hg_ref, t_ref, w_ref, b_ref, tn_ref):
    hg16 = hg_ref[...].astype(jnp.bfloat16)
    acc = jnp.dot(hg16, t_ref[...], preferred_element_type=jnp.float32)
    h = jnp.maximum(acc, 0.0).astype(jnp.bfloat16)
    tn = jnp.dot(h, w_ref[...], preferred_element_type=jnp.float32)
    tn_ref[...] = (tn + b_ref[...]).astype(jnp.bfloat16)


def _last_kernel(hg_ref, t_ref, out_ref):
    hg16 = hg_ref[...].astype(jnp.bfloat16)
    acc = jnp.dot(hg16, t_ref[...], preferred_element_type=jnp.float32)
    out_ref[...] = jnp.maximum(acc, 0.0)


def kernel(x, hg, W1, b1, W2, b2, W3, b3):
    t = pl.pallas_call(
        _xform_kernel,
        grid=(N // TILE_T,),
        in_specs=[
            pl.BlockSpec((TILE_T, D), lambda i: (i, 0)),
            pl.BlockSpec((D, D), lambda i: (0, 0)),
            pl.BlockSpec((1, D), lambda i: (0, 0)),
        ],
        out_specs=pl.BlockSpec((TILE_T, D), lambda i: (i, 0)),
        out_shape=jax.ShapeDtypeStruct((N, D), jnp.bfloat16),
        compiler_params=pltpu.CompilerParams(
            dimension_semantics=("parallel",)),
    )(x, W1, b1.reshape(1, D))

    for W, b in ((W2, b2), (W3, b3)):
        t = pl.pallas_call(
            _mid_kernel,
            grid=(N // TILE_M,),
            in_specs=[
                pl.BlockSpec((TILE_M, N), lambda i: (i, 0)),
                pl.BlockSpec((N, D), lambda i: (0, 0)),
                pl.BlockSpec((D, D), lambda i: (0, 0)),
                pl.BlockSpec((1, D), lambda i: (0, 0)),
            ],
            out_specs=pl.BlockSpec((TILE_M, D), lambda i: (i, 0)),
            out_shape=jax.ShapeDtypeStruct((N, D), jnp.bfloat16),
            compiler_params=pltpu.CompilerParams(
                dimension_semantics=("parallel",)),
        )(hg, t, W.astype(jnp.bfloat16), b.reshape(1, D))

    return pl.pallas_call(
        _last_kernel,
        grid=(N // TILE_M,),
        in_specs=[
            pl.BlockSpec((TILE_M, N), lambda i: (i, 0)),
            pl.BlockSpec((N, D), lambda i: (0, 0)),
        ],
        out_specs=pl.BlockSpec((TILE_M, D), lambda i: (i, 0)),
        out_shape=jax.ShapeDtypeStruct((N, D), jnp.float32),
        compiler_params=pltpu.CompilerParams(
            dimension_semantics=("parallel",)),
    )(hg, t)
